# Initial kernel scaffold; baseline (speedup 1.0000x reference)
import jax
import jax.numpy as jnp
from jax.experimental import pallas as pl


def kernel(x, edge_index, W1_0, b1_0, W2_0, b2_0, W1_1, b1_1, W2_1, b2_1, W1_2, b1_2, W2_2, b2_2, Wf, bf):
    raise NotImplementedError("write your pallas kernel here")


# speedup vs baseline: 9.0879x; 9.0879x over previous
import functools

import jax
import jax.numpy as jnp
from jax import lax
from jax.experimental import pallas as pl
from jax.experimental.pallas import tpu as pltpu
from jax.experimental.pallas import tpu_sc as plsc

N_NODES_K = 10000
N_EDGES_K = 320000
D_FEAT_K = 128
HID = 32

NW = 32
CH = 128
NCH = 80
EPW = CH * NCH
EPAD = EPW * NW
NPAD = 10240
RPS = NPAD // 16
NBUF = 4


def _sc_aggregate_build():
  mesh = plsc.VectorSubcoreMesh(core_axis_name="c", subcore_axis_name="s")

  @functools.partial(
      pl.kernel,
      mesh=mesh,
      out_type=jax.ShapeDtypeStruct((2, NPAD, HID), jnp.float32),
      compiler_params=pltpu.CompilerParams(use_tc_tiling_on_sc=False),
      scratch_types=[
          pltpu.VMEM((NCH, CH), jnp.int32),
          pltpu.VMEM((NCH, CH), jnp.int32),
          pltpu.VMEM((NBUF, CH, HID), jnp.float32),
          pltpu.VMEM_SHARED((NPAD, HID), jnp.float32),
          pltpu.SemaphoreType.DMA,
          pltpu.SemaphoreType.DMA,
          pltpu.SemaphoreType.DMA,
          pltpu.SemaphoreType.DMA,
      ],
  )
  def agg(y_hbm, src_hbm, dst_hbm, zeros_hbm, out_hbm,
          src_v, dst_v, rows_v, acc, sem0, sem1, sem2, sem3):
    sems = (sem0, sem1, sem2, sem3)
    c = lax.axis_index("c")
    s = lax.axis_index("s")
    wid = s * 2 + c

    pltpu.sync_copy(src_hbm.at[wid], src_v)
    pltpu.sync_copy(dst_hbm.at[wid], dst_v)
    pltpu.sync_copy(zeros_hbm.at[pl.ds(s * RPS, RPS)],
                    acc.at[pl.ds(s * RPS, RPS)])
    plsc.subcore_barrier()

    for b in range(NBUF):
      pltpu.async_copy(y_hbm.at[src_v.at[b]], rows_v.at[b], sems[b])

    def body(i, carry):
      for b in range(NBUF):
        j = i * NBUF + b
        pltpu.make_async_copy(y_hbm.at[src_v.at[j]], rows_v.at[b],
                              sems[b]).wait()
        pltpu.sync_copy(rows_v.at[b], acc.at[dst_v.at[j]], add=True)

        @pl.when(j + NBUF < NCH)
        def _():
          pltpu.async_copy(y_hbm.at[src_v.at[j + NBUF]], rows_v.at[b],
                           sems[b])
      return carry

    lax.fori_loop(0, NCH // NBUF, body, 0)
    plsc.subcore_barrier()

    pltpu.sync_copy(acc.at[pl.ds(s * RPS, RPS)],
                    out_hbm.at[c, pl.ds(s * RPS, RPS)])

  return agg


_sc_aggregate = _sc_aggregate_build()

BM = 1000
GRID = N_NODES_K // BM


def _proj_body(x_ref, w_ref, o_ref):
  o_ref[...] = jnp.dot(x_ref[...], w_ref[...],
                       preferred_element_type=jnp.float32,
                       precision=jax.lax.Precision.HIGHEST)


def _tc_project(x, w):
  m, k = x.shape
  n = w.shape[1]
  return pl.pallas_call(
      _proj_body,
      grid=(GRID,),
      in_specs=[
          pl.BlockSpec((BM, k), lambda i: (i, 0)),
          pl.BlockSpec((k, n), lambda i: (0, 0)),
      ],
      out_specs=pl.BlockSpec((BM, n), lambda i: (i, 0)),
      out_shape=jax.ShapeDtypeStruct((m, n), jnp.float32),
  )(x, w)


def _mlp_body(p_ref, y_ref, b1_ref, w2_ref, b2_ref, wn_ref, bn_ref, o_ref):
  agg = p_ref[0] + p_ref[1] + y_ref[...] + b1_ref[...]
  h = jnp.maximum(agg, 0.0)
  h = jnp.dot(h, w2_ref[...], preferred_element_type=jnp.float32,
              precision=jax.lax.Precision.HIGHEST) + b2_ref[...]
  h = jnp.maximum(h, 0.0)
  o_ref[...] = jnp.dot(h, wn_ref[...], preferred_element_type=jnp.float32,
                       precision=jax.lax.Precision.HIGHEST) + bn_ref[...]


def _tc_mlp(parts, y, b1, w2, b2, wn, bn):
  return pl.pallas_call(
      _mlp_body,
      grid=(GRID,),
      in_specs=[
          pl.BlockSpec((2, BM, HID), lambda i: (0, i, 0)),
          pl.BlockSpec((BM, HID), lambda i: (i, 0)),
          pl.BlockSpec((1, HID), lambda i: (0, 0)),
          pl.BlockSpec((HID, HID), lambda i: (0, 0)),
          pl.BlockSpec((1, HID), lambda i: (0, 0)),
          pl.BlockSpec((HID, HID), lambda i: (0, 0)),
          pl.BlockSpec((1, HID), lambda i: (0, 0)),
      ],
      out_specs=pl.BlockSpec((BM, HID), lambda i: (i, 0)),
      out_shape=jax.ShapeDtypeStruct((N_NODES_K, HID), jnp.float32),
  )(parts, y, b1, w2, b2, wn, bn)


def kernel(x, edge_index, W1_0, b1_0, W2_0, b2_0, W1_1, b1_1, W2_1, b2_1,
           W1_2, b1_2, W2_2, b2_2, Wf, bf):
  src = edge_index[0]
  dst = edge_index[1]
  pad = EPAD - N_EDGES_K
  src_p = jnp.concatenate(
      [src, jnp.zeros((pad,), jnp.int32)]).reshape(NW, NCH, CH)
  dst_p = jnp.concatenate(
      [dst, jnp.full((pad,), N_NODES_K, jnp.int32)]).reshape(NW, NCH, CH)
  zeros = jnp.zeros((NPAD, HID), jnp.float32)

  b1_0r = b1_0.reshape(1, HID)
  b2_0r = b2_0.reshape(1, HID)
  b1_1r = b1_1.reshape(1, HID)
  b2_1r = b2_1.reshape(1, HID)
  b1_2r = b1_2.reshape(1, HID)
  b2_2r = b2_2.reshape(1, HID)
  zeros_b = jnp.zeros((1, HID), jnp.float32)
  wf_pad = jnp.zeros((HID, HID), jnp.float32).at[:, :1].set(Wf)
  bf_pad = jnp.zeros((1, HID), jnp.float32).at[:, :1].set(
      bf.reshape(1, 1))

  y0 = _tc_project(x, W1_0)
  p0 = _sc_aggregate(y0, src_p, dst_p, zeros)
  y1 = _tc_mlp(p0, y0, b1_0r, W2_0, b2_0r, W1_1, zeros_b)
  p1 = _sc_aggregate(y1, src_p, dst_p, zeros)
  y2 = _tc_mlp(p1, y1, b1_1r, W2_1, b2_1r, W1_2, zeros_b)
  p2 = _sc_aggregate(y2, src_p, dst_p, zeros)
  out_wide = _tc_mlp(p2, y2, b1_2r, W2_2, b2_2r, wf_pad, bf_pad)
  return out_wide[:, :1]


# speedup vs baseline: 19.4811x; 2.1436x over previous
import functools

import jax
import jax.numpy as jnp
from jax import lax
from jax.experimental import pallas as pl
from jax.experimental.pallas import tpu as pltpu
from jax.experimental.pallas import tpu_sc as plsc

N_NODES_K = 10000
N_EDGES_K = 320000
D_FEAT_K = 128
HID = 32

NW = 32
CH = 128
NCH = 80
EPW = CH * NCH
EPAD = EPW * NW
NPAD = 10240
RPS = NPAD // 16
NBUF = 4


def _sc_aggregate_build():
  mesh = plsc.VectorSubcoreMesh(core_axis_name="c", subcore_axis_name="s")

  @functools.partial(
      pl.kernel,
      mesh=mesh,
      out_type=jax.ShapeDtypeStruct((2, NPAD, HID), jnp.float32),
      compiler_params=pltpu.CompilerParams(use_tc_tiling_on_sc=False),
      scratch_types=[
          pltpu.VMEM((NCH, CH), jnp.int32),
          pltpu.VMEM((NCH, CH), jnp.int32),
          pltpu.VMEM((NBUF, CH, HID), jnp.float32),
          pltpu.VMEM_SHARED((N_NODES_K, HID), jnp.float32),
          pltpu.VMEM_SHARED((NPAD, HID), jnp.float32),
          pltpu.SemaphoreType.DMA,
          pltpu.SemaphoreType.DMA,
          pltpu.SemaphoreType.DMA,
          pltpu.SemaphoreType.DMA,
      ],
  )
  def agg(y_hbm, src_hbm, dst_hbm, zeros_hbm, out_hbm,
          src_v, dst_v, rows_v, ytab, acc, sem0, sem1, sem2, sem3):
    sems = (sem0, sem1, sem2, sem3)
    c = lax.axis_index("c")
    s = lax.axis_index("s")
    wid = s * 2 + c

    pltpu.sync_copy(src_hbm.at[wid], src_v)
    pltpu.sync_copy(dst_hbm.at[wid], dst_v)
    tps = N_NODES_K // 16
    pltpu.sync_copy(y_hbm.at[pl.ds(s * tps, tps)], ytab.at[pl.ds(s * tps, tps)])
    pltpu.sync_copy(zeros_hbm.at[pl.ds(s * RPS, RPS)],
                    acc.at[pl.ds(s * RPS, RPS)])
    plsc.subcore_barrier()

    for b in range(NBUF):
      pltpu.async_copy(ytab.at[src_v.at[b]], rows_v.at[b], sems[b])

    def body(i, carry):
      for b in range(NBUF):
        j = i * NBUF + b
        pltpu.make_async_copy(ytab.at[src_v.at[j]], rows_v.at[b],
                              sems[b]).wait()
        pltpu.sync_copy(rows_v.at[b], acc.at[dst_v.at[j]], add=True)

        @pl.when(j + NBUF < NCH)
        def _():
          pltpu.async_copy(ytab.at[src_v.at[j + NBUF]], rows_v.at[b],
                           sems[b])
      return carry

    lax.fori_loop(0, NCH // NBUF, body, 0)
    plsc.subcore_barrier()

    pltpu.sync_copy(acc.at[pl.ds(s * RPS, RPS)],
                    out_hbm.at[c, pl.ds(s * RPS, RPS)])

  return agg


_sc_aggregate = _sc_aggregate_build()

BM = 1000
GRID = N_NODES_K // BM


def _proj_body(x_ref, w_ref, o_ref):
  o_ref[...] = jnp.dot(x_ref[...], w_ref[...],
                       preferred_element_type=jnp.float32)


def _tc_project(x, w):
  m, k = x.shape
  n = w.shape[1]
  return pl.pallas_call(
      _proj_body,
      grid=(GRID,),
      in_specs=[
          pl.BlockSpec((BM, k), lambda i: (i, 0)),
          pl.BlockSpec((k, n), lambda i: (0, 0)),
      ],
      out_specs=pl.BlockSpec((BM, n), lambda i: (i, 0)),
      out_shape=jax.ShapeDtypeStruct((m, n), jnp.float32),
  )(x, w)


def _mlp_body(p_ref, y_ref, b1_ref, w2_ref, b2_ref, wn_ref, bn_ref, o_ref):
  agg = p_ref[0] + p_ref[1] + y_ref[...] + b1_ref[...]
  h = jnp.maximum(agg, 0.0)
  h = jnp.dot(h, w2_ref[...], preferred_element_type=jnp.float32) + b2_ref[...]
  h = jnp.maximum(h, 0.0)
  o_ref[...] = jnp.dot(h, wn_ref[...], preferred_element_type=jnp.float32) + bn_ref[...]


def _tc_mlp(parts, y, b1, w2, b2, wn, bn):
  return pl.pallas_call(
      _mlp_body,
      grid=(GRID,),
      in_specs=[
          pl.BlockSpec((2, BM, HID), lambda i: (0, i, 0)),
          pl.BlockSpec((BM, HID), lambda i: (i, 0)),
          pl.BlockSpec((1, HID), lambda i: (0, 0)),
          pl.BlockSpec((HID, HID), lambda i: (0, 0)),
          pl.BlockSpec((1, HID), lambda i: (0, 0)),
          pl.BlockSpec((HID, HID), lambda i: (0, 0)),
          pl.BlockSpec((1, HID), lambda i: (0, 0)),
      ],
      out_specs=pl.BlockSpec((BM, HID), lambda i: (i, 0)),
      out_shape=jax.ShapeDtypeStruct((N_NODES_K, HID), jnp.float32),
  )(parts, y, b1, w2, b2, wn, bn)


def kernel(x, edge_index, W1_0, b1_0, W2_0, b2_0, W1_1, b1_1, W2_1, b2_1,
           W1_2, b1_2, W2_2, b2_2, Wf, bf):
  src = edge_index[0]
  dst = edge_index[1]
  pad = EPAD - N_EDGES_K
  src_p = jnp.concatenate(
      [src, jnp.zeros((pad,), jnp.int32)]).reshape(NW, NCH, CH)
  dst_p = jnp.concatenate(
      [dst, jnp.full((pad,), N_NODES_K, jnp.int32)]).reshape(NW, NCH, CH)
  zeros = jnp.zeros((NPAD, HID), jnp.float32)

  b1_0r = b1_0.reshape(1, HID)
  b2_0r = b2_0.reshape(1, HID)
  b1_1r = b1_1.reshape(1, HID)
  b2_1r = b2_1.reshape(1, HID)
  b1_2r = b1_2.reshape(1, HID)
  b2_2r = b2_2.reshape(1, HID)
  zeros_b = jnp.zeros((1, HID), jnp.float32)
  wf_pad = jnp.zeros((HID, HID), jnp.float32).at[:, :1].set(Wf)
  bf_pad = jnp.zeros((1, HID), jnp.float32).at[:, :1].set(
      bf.reshape(1, 1))

  y0 = _tc_project(x, W1_0)
  p0 = _sc_aggregate(y0, src_p, dst_p, zeros)
  y1 = _tc_mlp(p0, y0, b1_0r, W2_0, b2_0r, W1_1, zeros_b)
  p1 = _sc_aggregate(y1, src_p, dst_p, zeros)
  y2 = _tc_mlp(p1, y1, b1_1r, W2_1, b2_1r, W1_2, zeros_b)
  p2 = _sc_aggregate(y2, src_p, dst_p, zeros)
  out_wide = _tc_mlp(p2, y2, b1_2r, W2_2, b2_2r, wf_pad, bf_pad)
  return out_wide[:, :1]


# speedup vs baseline: 23.5524x; 1.2090x over previous
import functools

import jax
import jax.numpy as jnp
from jax import lax
from jax.experimental import pallas as pl
from jax.experimental.pallas import tpu as pltpu
from jax.experimental.pallas import tpu_sc as plsc

N_NODES_K = 10000
N_EDGES_K = 320000
HID = 32
PK = 128
NPK = 4

NW = 32
CH = 128
NCH = 80
EPW = CH * NCH
EPAD = EPW * NW
NPAD = 10240
NPAD4 = NPAD // NPK
RPS = NPAD // 16
RPS4 = NPAD4 // 16
NBUF = 4


def _sc_aggregate_build():
  mesh = plsc.VectorSubcoreMesh(core_axis_name="c", subcore_axis_name="s")

  @functools.partial(
      pl.kernel,
      mesh=mesh,
      out_type=jax.ShapeDtypeStruct((2, NPAD, HID), jnp.float32),
      compiler_params=pltpu.CompilerParams(use_tc_tiling_on_sc=False),
      scratch_types=[
          pltpu.VMEM((NCH, CH), jnp.int32),
          pltpu.VMEM((NCH, CH), jnp.int32),
          pltpu.VMEM((NBUF, CH, HID), jnp.float32),
          pltpu.VMEM_SHARED((NPAD, HID), jnp.float32),
          pltpu.VMEM_SHARED((NPAD, HID), jnp.float32),
          pltpu.SemaphoreType.DMA,
          pltpu.SemaphoreType.DMA,
          pltpu.SemaphoreType.DMA,
          pltpu.SemaphoreType.DMA,
      ],
  )
  def agg(y_hbm, src_hbm, dst_hbm, zeros_hbm, out_hbm,
          src_v, dst_v, rows_v, ytab, acc, sem0, sem1, sem2, sem3):
    sems = (sem0, sem1, sem2, sem3)
    c = lax.axis_index("c")
    s = lax.axis_index("s")
    wid = s * 2 + c

    pltpu.sync_copy(src_hbm.at[wid], src_v)
    pltpu.sync_copy(dst_hbm.at[wid], dst_v)
    pltpu.sync_copy(y_hbm.at[pl.ds(s * RPS, RPS)],
                    ytab.at[pl.ds(s * RPS, RPS)])
    pltpu.sync_copy(zeros_hbm.at[pl.ds(s * RPS, RPS)],
                    acc.at[pl.ds(s * RPS, RPS)])
    plsc.subcore_barrier()

    for b in range(NBUF):
      pltpu.async_copy(ytab.at[src_v.at[b]], rows_v.at[b], sems[b])

    def body(i, carry):
      for b in range(NBUF):
        j = i * NBUF + b
        pltpu.make_async_copy(ytab.at[src_v.at[j]], rows_v.at[b],
                              sems[b]).wait()
        pltpu.sync_copy(rows_v.at[b], acc.at[dst_v.at[j]], add=True)

        @pl.when(j + NBUF < NCH)
        def _():
          pltpu.async_copy(ytab.at[src_v.at[j + NBUF]], rows_v.at[b],
                           sems[b])
      return carry

    lax.fori_loop(0, NCH // NBUF, body, 0)
    plsc.subcore_barrier()

    pltpu.sync_copy(acc.at[pl.ds(s * RPS, RPS)],
                    out_hbm.at[c, pl.ds(s * RPS, RPS)])

  return agg


_sc_aggregate = _sc_aggregate_build()

BM4 = 256
GRID = NPAD4 // BM4


def _dot3(a, b):
  a_hi = a.astype(jnp.bfloat16)
  a_lo = (a - a_hi.astype(jnp.float32)).astype(jnp.bfloat16)
  b_hi = b.astype(jnp.bfloat16)
  b_lo = (b - b_hi.astype(jnp.float32)).astype(jnp.bfloat16)
  def d(u, v):
    return jnp.dot(u, v, preferred_element_type=jnp.float32)
  return d(a_hi, b_hi) + d(a_hi, b_lo) + d(a_lo, b_hi)


def _proj_body(x_ref, w_ref, o_ref):
  o_ref[...] = _dot3(x_ref[...], w_ref[...])


def _tc_project(x4, w1bd):
  return pl.pallas_call(
      _proj_body,
      grid=(GRID,),
      in_specs=[
          pl.BlockSpec((BM4, 4 * 128), lambda i: (i, 0)),
          pl.BlockSpec((4 * 128, PK), lambda i: (0, 0)),
      ],
      out_specs=pl.BlockSpec((BM4, PK), lambda i: (i, 0)),
      out_shape=jax.ShapeDtypeStruct((NPAD4, PK), jnp.float32),
  )(x4, w1bd)


def _mlp_body(p_ref, y_ref, b1_ref, w2_ref, b2_ref, wn_ref, bn_ref, o_ref):
  agg = p_ref[0] + p_ref[1] + y_ref[...] + b1_ref[...]
  h = jnp.maximum(agg, 0.0)
  h = _dot3(h, w2_ref[...]) + b2_ref[...]
  h = jnp.maximum(h, 0.0)
  o_ref[...] = _dot3(h, wn_ref[...]) + bn_ref[...]


def _tc_mlp(parts, y, b1t, w2bd, b2t, wnbd, bnt, n_out):
  return pl.pallas_call(
      _mlp_body,
      grid=(GRID,),
      in_specs=[
          pl.BlockSpec((2, BM4, PK), lambda i: (0, i, 0)),
          pl.BlockSpec((BM4, PK), lambda i: (i, 0)),
          pl.BlockSpec((1, PK), lambda i: (0, 0)),
          pl.BlockSpec((PK, PK), lambda i: (0, 0)),
          pl.BlockSpec((1, PK), lambda i: (0, 0)),
          pl.BlockSpec((PK, n_out), lambda i: (0, 0)),
          pl.BlockSpec((1, n_out), lambda i: (0, 0)),
      ],
      out_specs=pl.BlockSpec((BM4, n_out), lambda i: (i, 0)),
      out_shape=jax.ShapeDtypeStruct((NPAD4, n_out), jnp.float32),
  )(parts, y, b1t, w2bd, b2t, wnbd, bnt)


def _bd(w):
  return jnp.kron(jnp.eye(NPK, dtype=jnp.float32), w)


def _tile4(b):
  return jnp.tile(b, NPK).reshape(1, NPK * b.shape[0])


def kernel(x, edge_index, W1_0, b1_0, W2_0, b2_0, W1_1, b1_1, W2_1, b2_1,
           W1_2, b1_2, W2_2, b2_2, Wf, bf):
  src = edge_index[0]
  dst = edge_index[1]
  pad = EPAD - N_EDGES_K
  src_p = jnp.concatenate(
      [src, jnp.zeros((pad,), jnp.int32)]).reshape(NW, NCH, CH)
  dst_p = jnp.concatenate(
      [dst, jnp.full((pad,), N_NODES_K, jnp.int32)]).reshape(NW, NCH, CH)
  zeros = jnp.zeros((NPAD, HID), jnp.float32)

  x4 = x.reshape(N_NODES_K // NPK, NPK * 128)
  w1bd_0 = _bd(W1_0)
  w2bd_0, w1bd_1 = _bd(W2_0), _bd(W1_1)
  w2bd_1, w1bd_2 = _bd(W2_1), _bd(W1_2)
  w2bd_2 = _bd(W2_2)
  wfbd = jnp.kron(jnp.eye(NPK, dtype=jnp.float32), Wf)
  b1t_0, b2t_0 = _tile4(b1_0), _tile4(b2_0)
  b1t_1, b2t_1 = _tile4(b1_1), _tile4(b2_1)
  b1t_2, b2t_2 = _tile4(b1_2), _tile4(b2_2)
  bft = _tile4(bf)
  zeros_b = jnp.zeros((1, PK), jnp.float32)

  def agg_packed(y_packed):
    p = _sc_aggregate(y_packed.reshape(NPAD, HID), src_p, dst_p, zeros)
    return p.reshape(2, NPAD4, PK)

  y0 = _tc_project(x4, w1bd_0)
  p0 = agg_packed(y0)
  y1 = _tc_mlp(p0, y0, b1t_0, w2bd_0, b2t_0, w1bd_1, zeros_b, PK)
  p1 = agg_packed(y1)
  y2 = _tc_mlp(p1, y1, b1t_1, w2bd_1, b2t_1, w1bd_2, zeros_b, PK)
  p2 = agg_packed(y2)
  out_p = _tc_mlp(p2, y2, b1t_2, w2bd_2, b2t_2, wfbd, bft, NPK)
  return out_p[:N_NODES_K // NPK].reshape(N_NODES_K, 1)


# speedup vs baseline: 25.3936x; 1.0782x over previous
import functools

import jax
import jax.numpy as jnp
from jax import lax
from jax.experimental import pallas as pl
from jax.experimental.pallas import tpu as pltpu
from jax.experimental.pallas import tpu_sc as plsc

N_NODES_K = 10000
N_EDGES_K = 320000
HID = 32
PK = 128
NPK = 4

NW = 32
CH = 128
NCH = 80
EPW = CH * NCH
EPAD = EPW * NW
NPAD = 10240
NPAD4 = NPAD // NPK
RPS = NPAD // 16
RPS4 = NPAD4 // 16
NBUF = 8


def _sc_aggregate_build():
  mesh = plsc.VectorSubcoreMesh(core_axis_name="c", subcore_axis_name="s")

  @functools.partial(
      pl.kernel,
      mesh=mesh,
      out_type=jax.ShapeDtypeStruct((2, NPAD, HID), jnp.float32),
      compiler_params=pltpu.CompilerParams(use_tc_tiling_on_sc=False),
      scratch_types=[
          pltpu.VMEM((NCH, CH), jnp.int32),
          pltpu.VMEM((NCH, CH), jnp.int32),
          pltpu.VMEM((NBUF, CH, HID), jnp.float32),
          pltpu.VMEM_SHARED((NPAD, HID), jnp.float32),
          pltpu.VMEM_SHARED((NPAD, HID), jnp.float32),
          pltpu.SemaphoreType.DMA,
          pltpu.SemaphoreType.DMA,
          pltpu.SemaphoreType.DMA,
          pltpu.SemaphoreType.DMA,
          pltpu.SemaphoreType.DMA,
          pltpu.SemaphoreType.DMA,
          pltpu.SemaphoreType.DMA,
          pltpu.SemaphoreType.DMA,
      ],
  )
  def agg(y_hbm, src_hbm, dst_hbm, zeros_hbm, out_hbm,
          src_v, dst_v, rows_v, ytab, acc,
          sem0, sem1, sem2, sem3, sem4, sem5, sem6, sem7):
    sems = (sem0, sem1, sem2, sem3, sem4, sem5, sem6, sem7)
    c = lax.axis_index("c")
    s = lax.axis_index("s")
    wid = s * 2 + c

    c0 = pltpu.async_copy(src_hbm.at[wid], src_v, sem0)
    c1 = pltpu.async_copy(dst_hbm.at[wid], dst_v, sem1)
    c2 = pltpu.async_copy(y_hbm.at[pl.ds(s * RPS, RPS)],
                          ytab.at[pl.ds(s * RPS, RPS)], sem2)
    c3 = pltpu.async_copy(zeros_hbm.at[pl.ds(s * RPS, RPS)],
                          acc.at[pl.ds(s * RPS, RPS)], sem3)
    c0.wait(); c1.wait(); c2.wait(); c3.wait()
    plsc.subcore_barrier()

    for b in range(NBUF):
      pltpu.async_copy(ytab.at[src_v.at[b]], rows_v.at[b], sems[b])

    def body(i, carry):
      for b in range(NBUF):
        j = i * NBUF + b
        pltpu.make_async_copy(ytab.at[src_v.at[j]], rows_v.at[b],
                              sems[b]).wait()
        pltpu.sync_copy(rows_v.at[b], acc.at[dst_v.at[j]], add=True)

        @pl.when(j + NBUF < NCH)
        def _():
          pltpu.async_copy(ytab.at[src_v.at[j + NBUF]], rows_v.at[b],
                           sems[b])
      return carry

    lax.fori_loop(0, NCH // NBUF, body, 0)
    plsc.subcore_barrier()

    pltpu.sync_copy(acc.at[pl.ds(s * RPS, RPS)],
                    out_hbm.at[c, pl.ds(s * RPS, RPS)])

  return agg


_sc_aggregate = _sc_aggregate_build()

BM4 = 512
GRID = NPAD4 // BM4


def _dot3(a, b_hi, b_lo):
  a_hi = a.astype(jnp.bfloat16)
  a_lo = (a - a_hi.astype(jnp.float32)).astype(jnp.bfloat16)
  def d(u, v):
    return jnp.dot(u, v, preferred_element_type=jnp.float32)
  return d(a_hi, b_hi) + d(a_hi, b_lo) + d(a_lo, b_hi)


def _split(w):
  w_hi = w.astype(jnp.bfloat16)
  w_lo = (w - w_hi.astype(jnp.float32)).astype(jnp.bfloat16)
  return w_hi, w_lo


def _proj_body(x_ref, whi_ref, wlo_ref, o_ref):
  o_ref[...] = _dot3(x_ref[...], whi_ref[...], wlo_ref[...])


def _tc_project(x4, w_hi, w_lo):
  return pl.pallas_call(
      _proj_body,
      grid=(GRID,),
      in_specs=[
          pl.BlockSpec((BM4, 4 * 128), lambda i: (i, 0)),
          pl.BlockSpec((4 * 128, PK), lambda i: (0, 0)),
          pl.BlockSpec((4 * 128, PK), lambda i: (0, 0)),
      ],
      out_specs=pl.BlockSpec((BM4, PK), lambda i: (i, 0)),
      out_shape=jax.ShapeDtypeStruct((NPAD4, PK), jnp.float32),
  )(x4, w_hi, w_lo)


def _mlp_body(p_ref, y_ref, b1_ref, w2h_ref, w2l_ref, b2_ref,
              wnh_ref, wnl_ref, bn_ref, o_ref):
  agg = p_ref[0] + p_ref[1] + y_ref[...] + b1_ref[...]
  h = jnp.maximum(agg, 0.0)
  h = _dot3(h, w2h_ref[...], w2l_ref[...]) + b2_ref[...]
  h = jnp.maximum(h, 0.0)
  o_ref[...] = _dot3(h, wnh_ref[...], wnl_ref[...]) + bn_ref[...]


def _tc_mlp(parts, y, b1t, w2s, b2t, wns, bnt, n_out):
  return pl.pallas_call(
      _mlp_body,
      grid=(GRID,),
      in_specs=[
          pl.BlockSpec((2, BM4, PK), lambda i: (0, i, 0)),
          pl.BlockSpec((BM4, PK), lambda i: (i, 0)),
          pl.BlockSpec((1, PK), lambda i: (0, 0)),
          pl.BlockSpec((PK, PK), lambda i: (0, 0)),
          pl.BlockSpec((PK, PK), lambda i: (0, 0)),
          pl.BlockSpec((1, PK), lambda i: (0, 0)),
          pl.BlockSpec((PK, n_out), lambda i: (0, 0)),
          pl.BlockSpec((PK, n_out), lambda i: (0, 0)),
          pl.BlockSpec((1, n_out), lambda i: (0, 0)),
      ],
      out_specs=pl.BlockSpec((BM4, n_out), lambda i: (i, 0)),
      out_shape=jax.ShapeDtypeStruct((NPAD4, n_out), jnp.float32),
  )(parts, y, b1t, w2s[0], w2s[1], b2t, wns[0], wns[1], bnt)


def _bd(w):
  return jnp.kron(jnp.eye(NPK, dtype=jnp.float32), w)


def _tile4(b):
  return jnp.tile(b, NPK).reshape(1, NPK * b.shape[0])


def kernel(x, edge_index, W1_0, b1_0, W2_0, b2_0, W1_1, b1_1, W2_1, b2_1,
           W1_2, b1_2, W2_2, b2_2, Wf, bf):
  src = edge_index[0]
  dst = edge_index[1]
  pad = EPAD - N_EDGES_K
  src_p = jnp.concatenate(
      [src, jnp.zeros((pad,), jnp.int32)]).reshape(NW, NCH, CH)
  dst_p = jnp.concatenate(
      [dst, jnp.full((pad,), N_NODES_K, jnp.int32)]).reshape(NW, NCH, CH)
  zeros = jnp.zeros((NPAD, HID), jnp.float32)

  x4 = x.reshape(N_NODES_K // NPK, NPK * 128)
  w1bd_0 = _split(_bd(W1_0))
  w2bd_0, w1bd_1 = _split(_bd(W2_0)), _split(_bd(W1_1))
  w2bd_1, w1bd_2 = _split(_bd(W2_1)), _split(_bd(W1_2))
  w2bd_2 = _split(_bd(W2_2))
  wfbd = _split(jnp.kron(jnp.eye(NPK, dtype=jnp.float32), Wf))
  b1t_0, b2t_0 = _tile4(b1_0), _tile4(b2_0)
  b1t_1, b2t_1 = _tile4(b1_1), _tile4(b2_1)
  b1t_2, b2t_2 = _tile4(b1_2), _tile4(b2_2)
  bft = _tile4(bf)
  zeros_b = jnp.zeros((1, PK), jnp.float32)

  def agg_packed(y_packed):
    p = _sc_aggregate(y_packed.reshape(NPAD, HID), src_p, dst_p, zeros)
    return p.reshape(2, NPAD4, PK)

  y0 = _tc_project(x4, w1bd_0[0], w1bd_0[1])
  p0 = agg_packed(y0)
  y1 = _tc_mlp(p0, y0, b1t_0, w2bd_0, b2t_0, w1bd_1, zeros_b, PK)
  p1 = agg_packed(y1)
  y2 = _tc_mlp(p1, y1, b1t_1, w2bd_1, b2t_1, w1bd_2, zeros_b, PK)
  p2 = agg_packed(y2)
  out_p = _tc_mlp(p2, y2, b1t_2, w2bd_2, b2t_2, wfbd, bft, NPK)
  return out_p[:N_NODES_K // NPK].reshape(N_NODES_K, 1)


# speedup vs baseline: 27.9615x; 1.1011x over previous
import functools

import jax
import jax.numpy as jnp
from jax import lax
from jax.experimental import pallas as pl
from jax.experimental.pallas import tpu as pltpu
from jax.experimental.pallas import tpu_sc as plsc

N_NODES_K = 10000
N_EDGES_K = 320000
HID = 32
PK = 128
NPK = 4

NW = 32
CH = 128
NCT = N_EDGES_K // CH
NCHW = 80
NCTP = 2560
NPAD = 10240
NPAD4 = NPAD // NPK
RPS = NPAD // 16
RPS4 = NPAD4 // 16
NBUF = 8


def _sc_aggregate_build():
  mesh = plsc.VectorSubcoreMesh(core_axis_name="c", subcore_axis_name="s")

  @functools.partial(
      pl.kernel,
      mesh=mesh,
      out_type=jax.ShapeDtypeStruct((2, NPAD, HID), jnp.float32),
      compiler_params=pltpu.CompilerParams(use_tc_tiling_on_sc=False),
      scratch_types=[
          pltpu.VMEM((NCHW, CH), jnp.int32),
          pltpu.VMEM((NCHW, CH), jnp.int32),
          pltpu.VMEM((NBUF, CH, HID), jnp.float32),
          pltpu.VMEM_SHARED((NPAD, HID), jnp.float32),
          pltpu.VMEM_SHARED((NPAD, HID), jnp.float32),
          pltpu.SemaphoreType.DMA,
          pltpu.SemaphoreType.DMA,
          pltpu.SemaphoreType.DMA,
          pltpu.SemaphoreType.DMA,
          pltpu.SemaphoreType.DMA,
          pltpu.SemaphoreType.DMA,
          pltpu.SemaphoreType.DMA,
          pltpu.SemaphoreType.DMA,
      ],
  )
  def agg(y_hbm, edge_hbm, zeros_hbm, out_hbm,
          src_v, dst_v, rows_v, ytab, acc,
          sem0, sem1, sem2, sem3, sem4, sem5, sem6, sem7):
    sems = (sem0, sem1, sem2, sem3, sem4, sem5, sem6, sem7)
    c = lax.axis_index("c")
    s = lax.axis_index("s")
    wid = s * 2 + c

    ncw = jnp.where(wid < NW - 1, NCHW, NCT - (NW - 1) * NCHW)
    base = wid * NCHW

    c0 = pltpu.async_copy(edge_hbm.at[0, pl.ds(base, NCHW)], src_v, sem0)
    c1 = pltpu.async_copy(edge_hbm.at[1, pl.ds(base, NCHW)], dst_v, sem1)
    c2 = pltpu.async_copy(y_hbm.at[pl.ds(s * RPS, RPS)],
                          ytab.at[pl.ds(s * RPS, RPS)], sem2)
    c3 = pltpu.async_copy(zeros_hbm.at[pl.ds(s * RPS, RPS)],
                          acc.at[pl.ds(s * RPS, RPS)], sem3)
    c0.wait(); c1.wait(); c2.wait(); c3.wait()
    plsc.subcore_barrier()

    for b in range(NBUF):
      pltpu.async_copy(ytab.at[src_v.at[b]], rows_v.at[b], sems[b])

    def body(i, carry):
      for b in range(NBUF):
        j = i * NBUF + b

        @pl.when(j < ncw)
        def _():
          pltpu.make_async_copy(ytab.at[src_v.at[j]], rows_v.at[b],
                                sems[b]).wait()
          pltpu.sync_copy(rows_v.at[b], acc.at[dst_v.at[j]], add=True)

          @pl.when(j + NBUF < ncw)
          def _():
            pltpu.async_copy(ytab.at[src_v.at[j + NBUF]], rows_v.at[b],
                             sems[b])
      return carry

    lax.fori_loop(0, (NCHW + NBUF - 1) // NBUF, body, 0)
    plsc.subcore_barrier()

    pltpu.sync_copy(acc.at[pl.ds(s * RPS, RPS)],
                    out_hbm.at[c, pl.ds(s * RPS, RPS)])

  return agg


_sc_aggregate = _sc_aggregate_build()

BM4 = 512
GRID = NPAD4 // BM4


def _dot3(a, b_hi, b_lo):
  a_hi = a.astype(jnp.bfloat16)
  a_lo = (a - a_hi.astype(jnp.float32)).astype(jnp.bfloat16)
  def d(u, v):
    return jnp.dot(u, v, preferred_element_type=jnp.float32)
  return d(a_hi, b_hi) + d(a_hi, b_lo) + d(a_lo, b_hi)


def _split(w):
  w_hi = w.astype(jnp.bfloat16)
  w_lo = (w - w_hi.astype(jnp.float32)).astype(jnp.bfloat16)
  return w_hi, w_lo


def _proj_body(x_ref, whi_ref, wlo_ref, o_ref):
  o_ref[...] = _dot3(x_ref[...], whi_ref[...], wlo_ref[...])


def _tc_project(x4, w_hi, w_lo):
  return pl.pallas_call(
      _proj_body,
      grid=(GRID,),
      in_specs=[
          pl.BlockSpec((BM4, 4 * 128), lambda i: (i, 0)),
          pl.BlockSpec((4 * 128, PK), lambda i: (0, 0)),
          pl.BlockSpec((4 * 128, PK), lambda i: (0, 0)),
      ],
      out_specs=pl.BlockSpec((BM4, PK), lambda i: (i, 0)),
      out_shape=jax.ShapeDtypeStruct((NPAD4, PK), jnp.float32),
  )(x4, w_hi, w_lo)


def _mlp_body(p_ref, y_ref, b1_ref, w2h_ref, w2l_ref, b2_ref,
              wnh_ref, wnl_ref, bn_ref, o_ref):
  agg = p_ref[0] + p_ref[1] + y_ref[...] + b1_ref[...]
  h = jnp.maximum(agg, 0.0)
  h = _dot3(h, w2h_ref[...], w2l_ref[...]) + b2_ref[...]
  h = jnp.maximum(h, 0.0)
  o_ref[...] = _dot3(h, wnh_ref[...], wnl_ref[...]) + bn_ref[...]


def _tc_mlp(parts, y, b1t, w2s, b2t, wns, bnt, n_out):
  return pl.pallas_call(
      _mlp_body,
      grid=(GRID,),
      in_specs=[
          pl.BlockSpec((2, BM4, PK), lambda i: (0, i, 0)),
          pl.BlockSpec((BM4, PK), lambda i: (i, 0)),
          pl.BlockSpec((1, PK), lambda i: (0, 0)),
          pl.BlockSpec((PK, PK), lambda i: (0, 0)),
          pl.BlockSpec((PK, PK), lambda i: (0, 0)),
          pl.BlockSpec((1, PK), lambda i: (0, 0)),
          pl.BlockSpec((PK, n_out), lambda i: (0, 0)),
          pl.BlockSpec((PK, n_out), lambda i: (0, 0)),
          pl.BlockSpec((1, n_out), lambda i: (0, 0)),
      ],
      out_specs=pl.BlockSpec((BM4, n_out), lambda i: (i, 0)),
      out_shape=jax.ShapeDtypeStruct((NPAD4, n_out), jnp.float32),
  )(parts, y, b1t, w2s[0], w2s[1], b2t, wns[0], wns[1], bnt)


def _bd(w):
  return jnp.kron(jnp.eye(NPK, dtype=jnp.float32), w)


def _tile4(b):
  return jnp.tile(b, NPK).reshape(1, NPK * b.shape[0])


def kernel(x, edge_index, W1_0, b1_0, W2_0, b2_0, W1_1, b1_1, W2_1, b2_1,
           W1_2, b1_2, W2_2, b2_2, Wf, bf):
  edge3 = jnp.pad(edge_index.reshape(2, NCT, CH), ((0, 0), (0, NCTP - NCT),
                                                   (0, 0)))
  zeros = jnp.zeros((NPAD, HID), jnp.float32)

  x4 = x.reshape(N_NODES_K // NPK, NPK * 128)
  w1bd_0 = _split(_bd(W1_0))
  w2bd_0, w1bd_1 = _split(_bd(W2_0)), _split(_bd(W1_1))
  w2bd_1, w1bd_2 = _split(_bd(W2_1)), _split(_bd(W1_2))
  w2bd_2 = _split(_bd(W2_2))
  wfbd = _split(jnp.kron(jnp.eye(NPK, dtype=jnp.float32), Wf))
  b1t_0, b2t_0 = _tile4(b1_0), _tile4(b2_0)
  b1t_1, b2t_1 = _tile4(b1_1), _tile4(b2_1)
  b1t_2, b2t_2 = _tile4(b1_2), _tile4(b2_2)
  bft = _tile4(bf)
  zeros_b = jnp.zeros((1, PK), jnp.float32)

  def agg_packed(y_packed):
    p = _sc_aggregate(y_packed.reshape(NPAD, HID), edge3, zeros)
    return p.reshape(2, NPAD4, PK)

  y0 = _tc_project(x4, w1bd_0[0], w1bd_0[1])
  p0 = agg_packed(y0)
  y1 = _tc_mlp(p0, y0, b1t_0, w2bd_0, b2t_0, w1bd_1, zeros_b, PK)
  p1 = agg_packed(y1)
  y2 = _tc_mlp(p1, y1, b1t_1, w2bd_1, b2t_1, w1bd_2, zeros_b, PK)
  p2 = agg_packed(y2)
  out_p = _tc_mlp(p2, y2, b1t_2, w2bd_2, b2t_2, wfbd, bft, NPK)
  return out_p[:N_NODES_K // NPK].reshape(N_NODES_K, 1)
